# Initial kernel scaffold; baseline (speedup 1.0000x reference)
import functools

import jax
import jax.numpy as jnp
from jax import lax
from jax.experimental import pallas as pl
from jax.experimental.pallas import tpu as pltpu
from jax.experimental.pallas import tpu_sc as plsc

NUM_ROWS = 1000000
DIM = 32

NC = 2
NS = 16
NW = NC * NS

B = 16384 * 50
B_PER_W = B // NW
CHUNK = 1024
NCHUNK = B_PER_W // CHUNK


def _body(idx_hbm, table_hbm, out_hbm, idx_v, rows_v, sem):
    wid = lax.axis_index("s") * NC + lax.axis_index("c")
    base = wid * B_PER_W

    @pl.loop(0, NCHUNK)
    def _chunk(i):
        off = base + i * CHUNK
        pltpu.sync_copy(idx_hbm.at[pl.ds(off, CHUNK)], idx_v)
        pltpu.async_copy(table_hbm.at[idx_v], rows_v, sem).wait()
        pltpu.sync_copy(rows_v, out_hbm.at[pl.ds(off, CHUNK)])


@jax.jit
def _lookup(idx_flat, weight):
    mesh = plsc.VectorSubcoreMesh(core_axis_name="c", subcore_axis_name="s")
    f = functools.partial(
        pl.kernel,
        out_type=jax.ShapeDtypeStruct((B, DIM), jnp.float32),
        mesh=mesh,
        scratch_types=[
            pltpu.VMEM((CHUNK,), jnp.int32),
            pltpu.VMEM((CHUNK, DIM), jnp.float32),
            pltpu.SemaphoreType.DMA,
        ],
    )(_body)
    return f(idx_flat, weight)


def kernel(token_ids, weight):
    idx_flat = token_ids.reshape(-1).astype(jnp.int32)
    out = _lookup(idx_flat, weight)
    return out.reshape(token_ids.shape + (DIM,))


# speedup vs baseline: 1.0939x; 1.0939x over previous
import functools

import jax
import jax.numpy as jnp
from jax import lax
from jax.experimental import pallas as pl
from jax.experimental.pallas import tpu as pltpu
from jax.experimental.pallas import tpu_sc as plsc

NUM_ROWS = 1000000
DIM = 32

NC = 2
NS = 16
NW = NC * NS

B = 16384 * 50
B_PER_W = B // NW
CHUNK = 1024
NCHUNK = B_PER_W // CHUNK


def _body(idx_hbm, table_hbm, out_hbm, idx_v, rows_v, sem):
    wid = lax.axis_index("s") * NC + lax.axis_index("c")
    base = wid * B_PER_W

    @pl.loop(0, NCHUNK)
    def _chunk(i):
        off = base + i * CHUNK
        pltpu.sync_copy(idx_hbm.at[pl.ds(off, CHUNK)], idx_v)
        pltpu.async_copy(table_hbm.at[idx_v], rows_v, sem).wait()
        pltpu.sync_copy(rows_v, out_hbm.at[pl.ds(off, CHUNK)])


@jax.jit
def _lookup(idx_flat, weight):
    mesh = plsc.VectorSubcoreMesh(core_axis_name="c", subcore_axis_name="s")
    f = functools.partial(
        pl.kernel,
        out_type=jax.ShapeDtypeStruct((B, DIM), jnp.float32),
        mesh=mesh,
        scratch_types=[
            pltpu.VMEM((CHUNK,), jnp.int32),
            pltpu.VMEM((CHUNK, DIM), jnp.float32),
            pltpu.SemaphoreType.DMA,
        ],
        compiler_params=pltpu.CompilerParams(use_tc_tiling_on_sc=False),
    )(_body)
    return f(idx_flat, weight)


def kernel(token_ids, weight):
    idx_flat = token_ids.reshape(-1).astype(jnp.int32)
    out = _lookup(idx_flat, weight)
    return out.reshape(token_ids.shape + (DIM,))


# speedup vs baseline: 1.1122x; 1.0167x over previous
import functools

import jax
import jax.numpy as jnp
from jax import lax
from jax.experimental import pallas as pl
from jax.experimental.pallas import tpu as pltpu
from jax.experimental.pallas import tpu_sc as plsc

NUM_ROWS = 1000000
DIM = 32

NC = 2
NS = 16
NW = NC * NS

B = 16384 * 50
B_PER_W = B // NW
CHUNK = 640
NCHUNK = B_PER_W // CHUNK
NBUF = 4
LA = 2


def _body(idx_hbm, table_hbm, out_hbm, idx_v, rows_v, gsem, psem):
    wid = lax.axis_index("s") * NC + lax.axis_index("c")
    base = wid * B_PER_W

    pltpu.sync_copy(idx_hbm.at[pl.ds(base, B_PER_W)], idx_v)

    def gather(chunk, b):
        src = table_hbm.at[idx_v.at[pl.ds(chunk * CHUNK, CHUNK)]]
        return pltpu.make_async_copy(src, rows_v.at[b], gsem.at[b])

    def writeback(chunk, b):
        dst = out_hbm.at[pl.ds(base + chunk * CHUNK, CHUNK)]
        return pltpu.make_async_copy(rows_v.at[b], dst, psem.at[b])

    for b in range(LA):
        gather(b, b).start()

    @pl.loop(0, NCHUNK, step=NBUF)
    def _super(g0):
        for b in range(NBUF):
            g = g0 + b
            gather(g, b).wait()
            n = g + LA
            nb = (b + LA) % NBUF

            @pl.when(n < NCHUNK)
            def _():
                @pl.when(n >= NBUF)
                def _():
                    writeback(n - NBUF, nb).wait()

                gather(n, nb).start()

            writeback(g, b).start()

    for b in range(NBUF):
        writeback(0, b).wait()


@jax.jit
def _lookup(idx_flat, weight):
    mesh = plsc.VectorSubcoreMesh(core_axis_name="c", subcore_axis_name="s")
    f = functools.partial(
        pl.kernel,
        out_type=jax.ShapeDtypeStruct((B, DIM), jnp.float32),
        mesh=mesh,
        scratch_types=[
            pltpu.VMEM((B_PER_W,), jnp.int32),
            pltpu.VMEM((NBUF, CHUNK, DIM), jnp.float32),
            pltpu.SemaphoreType.DMA((NBUF,)),
            pltpu.SemaphoreType.DMA((NBUF,)),
        ],
        compiler_params=pltpu.CompilerParams(use_tc_tiling_on_sc=False),
    )(_body)
    return f(idx_flat, weight)


def kernel(token_ids, weight):
    idx_flat = token_ids.reshape(-1).astype(jnp.int32)
    out = _lookup(idx_flat, weight)
    return out.reshape(token_ids.shape + (DIM,))


# speedup vs baseline: 1.5017x; 1.3502x over previous
import functools

import jax
import jax.numpy as jnp
from jax import lax
from jax.experimental import pallas as pl
from jax.experimental.pallas import tpu as pltpu
from jax.experimental.pallas import tpu_sc as plsc

NUM_ROWS = 1000000
DIM = 32

NC = 2
NS = 16
NW = NC * NS

BATCH = 16384
SEQ = 50
BW = BATCH // NW
JV = BW // 16


def _body(idx_hbm, table_hbm, out_hbm, idx_v, rows_v, trans_v, gsem, psem):
    wid = lax.axis_index("s") * NC + lax.axis_index("c")
    b0 = wid * BW

    pltpu.sync_copy(idx_hbm.at[:, pl.ds(b0, BW)], idx_v)

    def gather(s, rb):
        src = table_hbm.at[idx_v.at[s]]
        return pltpu.make_async_copy(src, rows_v.at[rb], gsem.at[rb])

    def writeback(s, tb):
        dst = out_hbm.at[s, :, pl.ds(b0, BW)]
        return pltpu.make_async_copy(trans_v.at[tb], dst, psem.at[tb])

    iota = lax.iota(jnp.int32, 16)

    def transpose(rb, tb):
        rows = rows_v.at[rb]
        trans = trans_v.at[tb]

        @pl.loop(0, DIM)
        def _d(d):
            col = jnp.full((16,), 0, jnp.int32) + d
            for jv in range(JV):
                row = iota + (jv * 16)
                vec = plsc.load_gather(rows, [row, col])
                trans[d, pl.ds(jv * 16, 16)] = vec

    gather(0, 0).start()

    @pl.loop(0, SEQ, step=2)
    def _s2(s0):
        for h in range(2):
            s = s0 + h
            rb = h
            tb = h
            gather(s, rb).wait()

            @pl.when(s + 1 < SEQ)
            def _():
                gather(s + 1, 1 - rb).start()

            @pl.when(s >= 2)
            def _():
                writeback(s - 2, tb).wait()

            transpose(rb, tb)
            writeback(s, tb).start()

    for tb in range(2):
        writeback(0, tb).wait()


@jax.jit
def _lookup(idx_t, weight):
    mesh = plsc.VectorSubcoreMesh(core_axis_name="c", subcore_axis_name="s")
    f = functools.partial(
        pl.kernel,
        out_type=jax.ShapeDtypeStruct((SEQ, DIM, BATCH), jnp.float32),
        mesh=mesh,
        scratch_types=[
            pltpu.VMEM((SEQ, BW), jnp.int32),
            pltpu.VMEM((2, BW, DIM), jnp.float32),
            pltpu.VMEM((2, DIM, BW), jnp.float32),
            pltpu.SemaphoreType.DMA((2,)),
            pltpu.SemaphoreType.DMA((2,)),
        ],
        compiler_params=pltpu.CompilerParams(
            use_tc_tiling_on_sc=False, needs_layout_passes=False
        ),
    )(_body)
    return f(idx_t, weight)


def kernel(token_ids, weight):
    idx_t = token_ids.T.astype(jnp.int32)
    out = _lookup(idx_t, weight)
    return out.transpose(2, 0, 1)


# speedup vs baseline: 1.9614x; 1.3062x over previous
import functools

import jax
import jax.numpy as jnp
from jax import lax
from jax.experimental import pallas as pl
from jax.experimental.pallas import tpu as pltpu
from jax.experimental.pallas import tpu_sc as plsc

NUM_ROWS = 1000000
DIM = 32

NC = 2
NS = 16
NW = NC * NS

BATCH = 16384
SEQ = 50
BW = BATCH // NW
JV = BW // 16


def _body(idx_hbm, table_hbm, out_hbm, idx_v, rows_v, trans_v, gsem, psem):
    wid = lax.axis_index("s") * NC + lax.axis_index("c")
    b0 = wid * BW

    pltpu.sync_copy(idx_hbm.at[pl.ds(0, SEQ), pl.ds(b0, BW)], idx_v)

    def gather(s, rb):
        src = table_hbm.at[idx_v.at[s]]
        return pltpu.make_async_copy(src, rows_v.at[rb], gsem.at[rb])

    def writeback(s, tb):
        dst = out_hbm.at[s, :, pl.ds(b0, BW)]
        return pltpu.make_async_copy(trans_v.at[tb], dst, psem.at[tb])

    iota = lax.iota(jnp.int32, 16)

    def transpose(rb, tb):
        rows = rows_v.at[rb]
        trans = trans_v.at[tb]

        @plsc.parallel_loop(0, DIM, unroll=4)
        def _d(d):
            col = jnp.full((16,), 0, jnp.int32) + d
            for jv in range(JV):
                row = iota + (jv * 16)
                vec = plsc.load_gather(rows, [row, col])
                trans[d, pl.ds(jv * 16, 16)] = vec

    gather(0, 0).start()

    @pl.loop(0, SEQ, step=2)
    def _s2(s0):
        for h in range(2):
            s = s0 + h
            rb = h
            tb = h
            gather(s, rb).wait()

            @pl.when(s + 1 < SEQ)
            def _():
                gather(s + 1, 1 - rb).start()

            @pl.when(s >= 2)
            def _():
                writeback(s - 2, tb).wait()

            transpose(rb, tb)
            writeback(s, tb).start()

    for tb in range(2):
        writeback(0, tb).wait()


SEQ_PAD = 56


@jax.jit
def _lookup(idx_t, weight):
    mesh = plsc.VectorSubcoreMesh(core_axis_name="c", subcore_axis_name="s")
    f = functools.partial(
        pl.kernel,
        out_type=jax.ShapeDtypeStruct((SEQ, DIM, BATCH), jnp.float32),
        mesh=mesh,
        scratch_types=[
            pltpu.VMEM((SEQ, BW), jnp.int32),
            pltpu.VMEM((2, BW, DIM), jnp.float32),
            pltpu.VMEM((2, DIM, BW), jnp.float32),
            pltpu.SemaphoreType.DMA((2,)),
            pltpu.SemaphoreType.DMA((2,)),
        ],
        compiler_params=pltpu.CompilerParams(
            use_tc_tiling_on_sc=False, needs_layout_passes=False
        ),
    )(_body)
    return f(idx_t, weight)


def kernel(token_ids, weight):
    idx_t = token_ids.T.astype(jnp.int32)
    idx_p = jnp.pad(idx_t, ((0, SEQ_PAD - SEQ), (0, 0)))
    out = _lookup(idx_p, weight)
    return out.transpose(2, 0, 1)
